# Initial kernel scaffold; baseline (speedup 1.0000x reference)
import jax
import jax.numpy as jnp
from jax.experimental import pallas as pl


def kernel(hist_ndFeats, adj_values, W_gcn, W1, b1, W2, b2, edge_index, label_indices):
    raise NotImplementedError("write your pallas kernel here")


# speedup vs baseline: 2.9823x; 2.9823x over previous
import functools

import jax
import jax.numpy as jnp
from jax import lax
from jax.experimental import pallas as pl
from jax.experimental.pallas import tpu as pltpu
from jax.experimental.pallas import tpu_sc as plsc

_NC = 2
_NS = 16
_CHUNK = 128


def _spmm_body(n_nodes, dh, ct, x2_ref, src_ref, dst_ref, val_ref, out_ref,
               src_v, dst_v, val_v, rows, agg_sh):
    c = lax.axis_index("c")
    s = lax.axis_index("s")
    cbase = s * ct
    pltpu.sync_copy(src_ref.at[pl.ds(cbase, ct)], src_v)
    pltpu.sync_copy(dst_ref.at[pl.ds(cbase, ct)], dst_v)
    pltpu.sync_copy(val_ref.at[pl.ds(cbase, ct)], val_v)

    coff = c * n_nodes

    def adj(j, carry):
        for k in range(_CHUNK // 16):
            sl = pl.ds(k * 16, 16)
            src_v[j, sl] = src_v[j, sl] + coff
        return carry

    lax.fori_loop(0, ct, adj, 0)

    def zloop(i, carry):
        for k in range(dh // 16):
            rows[i, pl.ds(k * 16, 16)] = jnp.zeros((16,), jnp.float32)
        return carry

    lax.fori_loop(0, _CHUNK, zloop, 0)

    zrows = n_nodes // _NS
    zbase = s * zrows
    nfull, rem = zrows // _CHUNK, zrows % _CHUNK
    for m in range(nfull):
        pltpu.sync_copy(rows.at[pl.ds(0, _CHUNK)],
                        agg_sh.at[pl.ds(zbase + m * _CHUNK, _CHUNK)])
    if rem:
        pltpu.sync_copy(rows.at[pl.ds(0, rem)],
                        agg_sh.at[pl.ds(zbase + nfull * _CHUNK, rem)])
    plsc.subcore_barrier()

    def chunk(j, carry):
        pltpu.sync_copy(x2_ref.at[src_v.at[j]], rows)

        def scale(g, icarry):
            vv = val_v[j, pl.ds(g * 16, 16)]
            for e16 in range(16):
                bc = lax.broadcast(vv[e16], (16,))
                e = g * 16 + e16
                for k in range(dh // 16):
                    sl = pl.ds(k * 16, 16)
                    rows[e, sl] = rows[e, sl] * bc
            return icarry

        lax.fori_loop(0, _CHUNK // 16, scale, 0)
        pltpu.sync_copy(rows, agg_sh.at[dst_v.at[j]], add=True)
        return carry

    lax.fori_loop(0, ct, chunk, 0)
    plsc.subcore_barrier()

    pltpu.sync_copy(agg_sh.at[pl.ds(zbase, zrows)],
                    out_ref.at[pl.ds(c * n_nodes + zbase, zrows)])


def _spmm(x2, srcm, dstm, valm, n_nodes, dh, ct):
    body = functools.partial(_spmm_body, n_nodes, dh, ct)
    return pl.kernel(
        body,
        out_type=jax.ShapeDtypeStruct((2 * n_nodes, dh), jnp.float32),
        mesh=plsc.VectorSubcoreMesh(core_axis_name="c", subcore_axis_name="s",
                                    num_cores=_NC, num_subcores=_NS),
        scratch_types=[
            pltpu.VMEM((ct, _CHUNK), jnp.int32),
            pltpu.VMEM((ct, _CHUNK), jnp.int32),
            pltpu.VMEM((ct, _CHUNK), jnp.float32),
            pltpu.VMEM((_CHUNK, dh), jnp.float32),
            pltpu.VMEM_SHARED((n_nodes, dh), jnp.float32),
        ],
    )(x2, srcm, dstm, valm)


def _dense_body(a0_ref, a1_ref, wg0_ref, wg1_ref, w1_ref, b1_ref, w2_ref,
                b2_ref, out_ref):
    z = jnp.dot(a0_ref[...], wg0_ref[...], preferred_element_type=jnp.float32)
    z = z + jnp.dot(a1_ref[...], wg1_ref[...], preferred_element_type=jnp.float32)
    h = jnp.maximum(z, 0.0)
    t = jnp.dot(h, w1_ref[...], preferred_element_type=jnp.float32) + b1_ref[...]
    u = jnp.maximum(t, 0.0)
    out_ref[...] = (jnp.dot(u, w2_ref[...], preferred_element_type=jnp.float32)
                    + b2_ref[...])


def _dense_chain(agg0, agg1, wg0, wg1, w1, b1, w2, b2, block_rows):
    n, dh = agg0.shape
    d_out = w2.shape[1]
    grid = n // block_rows
    full = lambda i: (0, 0)
    return pl.pallas_call(
        _dense_body,
        grid=(grid,),
        in_specs=[
            pl.BlockSpec((block_rows, dh), lambda i: (i, 0)),
            pl.BlockSpec((block_rows, dh), lambda i: (i, 0)),
            pl.BlockSpec(wg0.shape, full),
            pl.BlockSpec(wg1.shape, full),
            pl.BlockSpec(w1.shape, full),
            pl.BlockSpec(b1.shape, full),
            pl.BlockSpec(w2.shape, full),
            pl.BlockSpec(b2.shape, full),
        ],
        out_specs=pl.BlockSpec((block_rows, d_out), lambda i: (i, 0)),
        out_shape=jax.ShapeDtypeStruct((n, d_out), jnp.float32),
    )(agg0, agg1, wg0, wg1, w1, b1, w2, b2)


def _gather_body(lt, tab_len, x_ref, lab_ref, out_ref, lab_v, tab_v, out_v):
    c = lax.axis_index("c")
    s = lax.axis_index("s")
    wid = s * _NC + c
    pltpu.sync_copy(x_ref, tab_v)
    pltpu.sync_copy(lab_ref.at[pl.ds(wid * lt, lt)], lab_v)

    def loop(v, carry):
        idx = lab_v[pl.ds(v * 16, 16)]
        i2 = idx * 2
        a = plsc.load_gather(tab_v, [i2])
        b = plsc.load_gather(tab_v, [i2 + 1])
        si = lax.iota(jnp.int32, 16) * 2 + v * 32
        plsc.store_scatter(out_v, [si], a)
        plsc.store_scatter(out_v, [si + 1], b)
        return carry

    lax.fori_loop(0, lt // 16, loop, 0)
    pltpu.sync_copy(out_v, out_ref.at[pl.ds(wid * 2 * lt, 2 * lt)])


def _label_gather(pred_flat, labels_p, lt):
    lp = labels_p.shape[0]
    tab_len = pred_flat.shape[0]
    body = functools.partial(_gather_body, lt, tab_len)
    return pl.kernel(
        body,
        out_type=jax.ShapeDtypeStruct((2 * lp,), jnp.float32),
        mesh=plsc.VectorSubcoreMesh(core_axis_name="c", subcore_axis_name="s",
                                    num_cores=_NC, num_subcores=_NS),
        scratch_types=[
            pltpu.VMEM((lt,), jnp.int32),
            pltpu.VMEM((tab_len,), jnp.float32),
            pltpu.VMEM((2 * lt,), jnp.float32),
        ],
        compiler_params=pltpu.CompilerParams(needs_layout_passes=False),
    )(pred_flat, labels_p)


def kernel(hist_ndFeats, adj_values, W_gcn, W1, b1, W2, b2, edge_index,
           label_indices):
    n, d = hist_ndFeats.shape
    dh = d // 2
    e = edge_index.shape[1]
    l = label_indices.shape[0]

    gran = _NS * _CHUNK * 8
    ep = ((e + gran - 1) // gran) * gran
    pad = ep - e
    src = jnp.pad(edge_index[0], (0, pad)).reshape(ep // _CHUNK, _CHUNK)
    dst = jnp.pad(edge_index[1], (0, pad)).reshape(ep // _CHUNK, _CHUNK)
    val = jnp.pad(adj_values, (0, pad)).reshape(ep // _CHUNK, _CHUNK)
    ct = ep // _CHUNK // _NS
    n_p = ((n + 127) // 128) * 128
    xp = jnp.pad(hist_ndFeats, ((0, n_p - n), (0, 0)))
    x2 = jnp.concatenate([xp[:, :dh], xp[:, dh:]], axis=0)

    agg2 = _spmm(x2, src, dst, val, n_p, dh, ct)
    agg0, agg1 = agg2[:n], agg2[n_p:n_p + n]

    pred_all = _dense_chain(agg0, agg1, W_gcn[:dh], W_gcn[dh:], W1,
                            b1.reshape(1, -1), W2, b2.reshape(1, -1),
                            block_rows=1000)

    lt = 320
    lp = ((l + _NC * _NS * lt - 1) // (_NC * _NS * lt)) * (_NC * _NS * lt)
    labels_p = jnp.pad(label_indices, (0, lp - l))
    out_flat = _label_gather(pred_all.reshape(-1), labels_p, lt)
    return out_flat.reshape(lp, 2)[:l]


# speedup vs baseline: 3.6883x; 1.2367x over previous
import functools

import jax
import jax.numpy as jnp
from jax import lax
from jax.experimental import pallas as pl
from jax.experimental.pallas import tpu as pltpu
from jax.experimental.pallas import tpu_sc as plsc

_NC = 2
_NS = 16
_CHUNK = 64


_NBUF = 4
_GRP = 8
_RSLOTS = 3


def _spmm_body(n_nodes, dh, ct, x2_ref, src2_ref, dst_ref, val_ref, out_ref,
               src_r, dst_r, val_r, rows_bufs, agg_sh, gsem, ssem, esem):
    rows = rows_bufs[0]
    c = lax.axis_index("c")
    s = lax.axis_index("s")
    cbase = s * ct
    rring = _RSLOTS * _GRP

    def zloop(i, carry):
        for k in range(dh // 16):
            rows[i, pl.ds(k * 16, 16)] = jnp.zeros((16,), jnp.float32)
        return carry

    lax.fori_loop(0, _CHUNK, zloop, 0)

    zrows = n_nodes // _NS
    zbase = s * zrows
    nfull, rem = zrows // _CHUNK, zrows % _CHUNK
    for m in range(nfull):
        pltpu.sync_copy(rows.at[pl.ds(0, _CHUNK)],
                        agg_sh.at[pl.ds(zbase + m * _CHUNK, _CHUNK)])
    if rem:
        pltpu.sync_copy(rows.at[pl.ds(0, rem)],
                        agg_sh.at[pl.ds(zbase + nfull * _CHUNK, rem)])

    pltpu.sync_copy(src2_ref.at[c, pl.ds(cbase, _GRP)], src_r.at[pl.ds(0, _GRP)])
    pltpu.sync_copy(dst_ref.at[pl.ds(cbase, _GRP)], dst_r.at[pl.ds(0, _GRP)])
    pltpu.sync_copy(val_ref.at[pl.ds(cbase, _GRP)], val_r.at[pl.ds(0, _GRP)])
    plsc.subcore_barrier()

    def issue_gather(rj, buf):
        pltpu.async_copy(x2_ref.at[src_r.at[rj]], buf, gsem)

    def wait_gather(buf):
        pltpu.make_async_copy(x2_ref.at[src_r.at[0]], buf, gsem).wait()

    def issue_scatter(rj, buf):
        pltpu.async_copy(buf, agg_sh.at[dst_r.at[rj]], ssem, add=True)

    def wait_scatter(buf):
        pltpu.make_async_copy(buf, agg_sh.at[dst_r.at[0]], ssem).wait()

    def scale_buf(rj, buf):
        def scale(g, icarry):
            vv = val_r[rj, pl.ds(g * 16, 16)]
            for e16 in range(16):
                bc = lax.broadcast(vv[e16], (16,))
                e = g * 16 + e16
                for k in range(dh // 16):
                    sl = pl.ds(k * 16, 16)
                    buf[e, sl] = buf[e, sl] * bc
            return icarry

        lax.fori_loop(0, _CHUNK // 16, scale, 0)

    issue_gather(0, rows_bufs[0])

    def pipe(k, carry):
        for b in range(_NBUF):
            j = _NBUF * k + b
            buf = rows_bufs[b]
            nxt = rows_bufs[(b + 1) % _NBUF]

            @pl.when(jnp.logical_and(j % _GRP == 0, j + _GRP < ct))
            def _():
                hb = pl.multiple_of(cbase + j + _GRP, _GRP)
                rb = pl.multiple_of((j + _GRP) % rring, _GRP)
                pltpu.async_copy(src2_ref.at[c, pl.ds(hb, _GRP)],
                                 src_r.at[pl.ds(rb, _GRP)], esem)
                pltpu.async_copy(dst_ref.at[pl.ds(hb, _GRP)],
                                 dst_r.at[pl.ds(rb, _GRP)], esem)
                pltpu.async_copy(val_ref.at[pl.ds(hb, _GRP)],
                                 val_r.at[pl.ds(rb, _GRP)], esem)

            @pl.when(j >= _NBUF - 1)
            def _():
                wait_scatter(nxt)

            @pl.when(jnp.logical_and((j + 1) % _GRP == 0, j + 1 < ct))
            def _():
                pltpu.make_async_copy(src2_ref.at[0, pl.ds(0, _GRP)],
                                      src_r.at[pl.ds(0, _GRP)], esem).wait()
                pltpu.make_async_copy(dst_ref.at[pl.ds(0, _GRP)],
                                      dst_r.at[pl.ds(0, _GRP)], esem).wait()
                pltpu.make_async_copy(val_ref.at[pl.ds(0, _GRP)],
                                      val_r.at[pl.ds(0, _GRP)], esem).wait()

            @pl.when(j + 1 < ct)
            def _():
                issue_gather((j + 1) % rring, nxt)

            wait_gather(buf)
            scale_buf(j % rring, buf)
            issue_scatter(j % rring, buf)
        return carry

    lax.fori_loop(0, ct // _NBUF, pipe, 0)
    for j in range(ct - _NBUF + 1, ct):
        wait_scatter(rows_bufs[j % _NBUF])
    plsc.subcore_barrier()

    pltpu.sync_copy(agg_sh.at[pl.ds(zbase, zrows)],
                    out_ref.at[pl.ds(c * n_nodes + zbase, zrows)])


def _spmm(x2, src2, dstm, valm, n_nodes, dh, ct):
    body = functools.partial(_spmm_body, n_nodes, dh, ct)
    rring = _RSLOTS * _GRP
    return pl.kernel(
        body,
        out_type=jax.ShapeDtypeStruct((2 * n_nodes, dh), jnp.float32),
        mesh=plsc.VectorSubcoreMesh(core_axis_name="c", subcore_axis_name="s",
                                    num_cores=_NC, num_subcores=_NS),
        scratch_types=[
            pltpu.VMEM((rring, _CHUNK), jnp.int32),
            pltpu.VMEM((rring, _CHUNK), jnp.int32),
            pltpu.VMEM((rring, _CHUNK), jnp.float32),
            [pltpu.VMEM((_CHUNK, dh), jnp.float32) for _ in range(_NBUF)],
            pltpu.VMEM_SHARED((n_nodes, dh), jnp.float32),
            pltpu.SemaphoreType.DMA,
            pltpu.SemaphoreType.DMA,
            pltpu.SemaphoreType.DMA,
        ],
    )(x2, src2, dstm, valm)


def _dense_body(a0_ref, a1_ref, wg0_ref, wg1_ref, w1_ref, b1_ref, w2_ref,
                b2_ref, out_ref):
    z = jnp.dot(a0_ref[...], wg0_ref[...], preferred_element_type=jnp.float32)
    z = z + jnp.dot(a1_ref[...], wg1_ref[...], preferred_element_type=jnp.float32)
    h = jnp.maximum(z, 0.0)
    t = jnp.dot(h, w1_ref[...], preferred_element_type=jnp.float32) + b1_ref[...]
    u = jnp.maximum(t, 0.0)
    out_ref[...] = (jnp.dot(u, w2_ref[...], preferred_element_type=jnp.float32)
                    + b2_ref[...])


def _dense_chain(agg0, agg1, wg0, wg1, w1, b1, w2, b2, block_rows):
    n, dh = agg0.shape
    d_out = w2.shape[1]
    grid = n // block_rows
    full = lambda i: (0, 0)
    return pl.pallas_call(
        _dense_body,
        grid=(grid,),
        in_specs=[
            pl.BlockSpec((block_rows, dh), lambda i: (i, 0)),
            pl.BlockSpec((block_rows, dh), lambda i: (i, 0)),
            pl.BlockSpec(wg0.shape, full),
            pl.BlockSpec(wg1.shape, full),
            pl.BlockSpec(w1.shape, full),
            pl.BlockSpec(b1.shape, full),
            pl.BlockSpec(w2.shape, full),
            pl.BlockSpec(b2.shape, full),
        ],
        out_specs=pl.BlockSpec((block_rows, d_out), lambda i: (i, 0)),
        out_shape=jax.ShapeDtypeStruct((n, d_out), jnp.float32),
    )(agg0, agg1, wg0, wg1, w1, b1, w2, b2)


def _gather_body(lt, tab_len, x_ref, lab_ref, out_ref, lab_v, tab_v, out_v):
    c = lax.axis_index("c")
    s = lax.axis_index("s")
    wid = s * _NC + c
    pltpu.sync_copy(x_ref, tab_v)
    pltpu.sync_copy(lab_ref.at[pl.ds(wid * lt, lt)], lab_v)

    def loop(v, carry):
        idx = lab_v[pl.ds(v * 16, 16)]
        i2 = idx * 2
        a = plsc.load_gather(tab_v, [i2])
        b = plsc.load_gather(tab_v, [i2 + 1])
        si = lax.iota(jnp.int32, 16) * 2 + v * 32
        plsc.store_scatter(out_v, [si], a)
        plsc.store_scatter(out_v, [si + 1], b)
        return carry

    lax.fori_loop(0, lt // 16, loop, 0)
    pltpu.sync_copy(out_v, out_ref.at[pl.ds(wid * 2 * lt, 2 * lt)])


def _label_gather(pred_flat, labels_p, lt):
    lp = labels_p.shape[0]
    tab_len = pred_flat.shape[0]
    body = functools.partial(_gather_body, lt, tab_len)
    return pl.kernel(
        body,
        out_type=jax.ShapeDtypeStruct((2 * lp,), jnp.float32),
        mesh=plsc.VectorSubcoreMesh(core_axis_name="c", subcore_axis_name="s",
                                    num_cores=_NC, num_subcores=_NS),
        scratch_types=[
            pltpu.VMEM((lt,), jnp.int32),
            pltpu.VMEM((tab_len,), jnp.float32),
            pltpu.VMEM((2 * lt,), jnp.float32),
        ],
        compiler_params=pltpu.CompilerParams(needs_layout_passes=False),
    )(pred_flat, labels_p)


def kernel(hist_ndFeats, adj_values, W_gcn, W1, b1, W2, b2, edge_index,
           label_indices):
    n, d = hist_ndFeats.shape
    dh = d // 2
    e = edge_index.shape[1]
    l = label_indices.shape[0]

    gran = _NS * _CHUNK * 8
    ep = ((e + gran - 1) // gran) * gran
    pad = ep - e
    n_p = ((n + 127) // 128) * 128
    src = jnp.pad(edge_index[0], (0, pad)).reshape(ep // _CHUNK, _CHUNK)
    src2 = jnp.stack([src, src + n_p])
    dst = jnp.pad(edge_index[1], (0, pad)).reshape(ep // _CHUNK, _CHUNK)
    val = jnp.pad(adj_values, (0, pad)).reshape(ep // _CHUNK, _CHUNK)
    ct = ep // _CHUNK // _NS
    xp = jnp.pad(hist_ndFeats, ((0, n_p - n), (0, 0)))
    x2 = jnp.concatenate([xp[:, :dh], xp[:, dh:]], axis=0)

    agg2 = _spmm(x2, src2, dst, val, n_p, dh, ct)
    agg0, agg1 = agg2[:n], agg2[n_p:n_p + n]

    pred_all = _dense_chain(agg0, agg1, W_gcn[:dh], W_gcn[dh:], W1,
                            b1.reshape(1, -1), W2, b2.reshape(1, -1),
                            block_rows=1000)

    lt = 320
    lp = ((l + _NC * _NS * lt - 1) // (_NC * _NS * lt)) * (_NC * _NS * lt)
    labels_p = jnp.pad(label_indices, (0, lp - l))
    out_flat = _label_gather(pred_all.reshape(-1), labels_p, lt)
    return out_flat.reshape(lp, 2)[:l]
